# baseline (speedup 1.0000x reference)
import functools
import math

import jax
import jax.numpy as jnp
from jax import lax
from jax.experimental import pallas as pl
from jax.experimental.pallas import tpu as pltpu
from jax.experimental.pallas import tpu_sc as plsc

N_RBF = 50
CUTOFF = 10.0
RBF_PAD = 64
H = 128
NCORES = 2
NSUB = 16
NW = NCORES * NSUB
CH = 80
GRP = 5
VEC = 16

_SC_PARAMS = pltpu.CompilerParams(needs_layout_passes=False)


def _splat_i32(v):
    return jnp.full((VEC,), v, jnp.int32)


def _mesh():
    return plsc.VectorSubcoreMesh(core_axis_name="c", subcore_axis_name="s")


def _sc_gather_geom(F, Cflat, row3d, col3d):
    rows_w = row3d.shape[1]
    per_w = rows_w * CH
    E = NW * per_w
    NP4 = Cflat.shape[0]
    out_sds = jax.ShapeDtypeStruct((E, H), jnp.float32)

    @functools.partial(
        pl.kernel, mesh=_mesh(),
        out_type=[out_sds, out_sds,
                  jax.ShapeDtypeStruct((E * 4,), jnp.float32)],
        compiler_params=_SC_PARAMS,
        scratch_types=[
            pltpu.VMEM((rows_w, CH), jnp.int32),
            pltpu.VMEM((rows_w, CH), jnp.int32),
            pltpu.VMEM((GRP * CH, H), jnp.float32),
            pltpu.VMEM((NP4,), jnp.float32),
            pltpu.VMEM((GRP * CH * 4,), jnp.float32),
        ],
    )
    def k(F_hbm, C_hbm, row_hbm, col_hbm, orow_hbm, ocol_hbm, ed_hbm,
          ridx_v, cidx_v, buf, Cv, ebuf):
        c = lax.axis_index("c")
        s = lax.axis_index("s")
        wid = c * NSUB + s
        base = wid * per_w
        pltpu.sync_copy(row_hbm.at[wid], ridx_v)
        pltpu.sync_copy(col_hbm.at[wid], cidx_v)
        pltpu.sync_copy(C_hbm, Cv)

        def do(idx_v, out_hbm):
            @pl.loop(0, rows_w // GRP)
            def _(g):
                for j in range(GRP):
                    pltpu.sync_copy(F_hbm.at[idx_v.at[g * GRP + j]],
                                    buf.at[pl.ds(j * CH, CH)])
                pltpu.sync_copy(buf, out_hbm.at[pl.ds(base + g * GRP * CH,
                                                      GRP * CH)])

        do(ridx_v, orow_hbm)
        do(cidx_v, ocol_hbm)

        @pl.loop(0, GRP * CH * 4 // VEC)
        def _(z):
            ebuf[pl.ds(z * VEC, VEC)] = jnp.zeros((VEC,), jnp.float32)

        iota = lax.iota(jnp.int32, VEC)

        @pl.loop(0, rows_w // GRP)
        def _(g):
            for j in range(GRP):
                for kk in range(CH // VEC):
                    sl = pl.ds(kk * VEC, VEC)
                    rv4 = ridx_v[g * GRP + j, sl] * 4
                    cv4 = cidx_v[g * GRP + j, sl] * 4
                    flat = (iota + j * CH + kk * VEC) * 4
                    for comp in range(3):
                        a = plsc.load_gather(Cv, [rv4 + comp])
                        b = plsc.load_gather(Cv, [cv4 + comp])
                        plsc.store_scatter(ebuf, [flat + comp], a - b)
            pltpu.sync_copy(ebuf,
                            ed_hbm.at[pl.ds((wid * per_w + g * GRP * CH) * 4,
                                            GRP * CH * 4)])

    return k(F, Cflat, row3d, col3d)


def _sc_scatter(m, cu_flat, row3d, zeros):
    rows_w = row3d.shape[1]
    per_w = rows_w * CH
    NP = zeros.shape[0]
    NR = NP // 8
    n_sub = NP // NSUB
    nr_sub = NR // NSUB

    @functools.partial(
        pl.kernel, mesh=_mesh(),
        out_type=[jax.ShapeDtypeStruct((NCORES, NP, H), jnp.float32),
                  jax.ShapeDtypeStruct((NCORES, NR, H), jnp.float32)],
        compiler_params=_SC_PARAMS,
        scratch_types=[
            pltpu.VMEM((rows_w, CH), jnp.int32),
            pltpu.VMEM((GRP * CH * 4,), jnp.float32),
            pltpu.VMEM((CH, H), jnp.float32),
            pltpu.VMEM((CH, H), jnp.float32),
            pltpu.VMEM((CH,), jnp.int32),
            pltpu.VMEM_SHARED((NP, H), jnp.float32),
            pltpu.VMEM_SHARED((NR, H), jnp.float32),
        ],
    )
    def k(m_hbm, cu_hbm, row_hbm, zeros_hbm, fout_hbm, cout_hbm,
          ridx_v, cubuf, mbuf, slot, idx2, facc, cacc):
        c = lax.axis_index("c")
        s = lax.axis_index("s")
        wid = c * NSUB + s
        base = wid * per_w
        pltpu.sync_copy(zeros_hbm.at[pl.ds(s * n_sub, n_sub)],
                        facc.at[pl.ds(s * n_sub, n_sub)])
        pltpu.sync_copy(zeros_hbm.at[pl.ds(s * nr_sub, nr_sub)],
                        cacc.at[pl.ds(s * nr_sub, nr_sub)])
        pltpu.sync_copy(zeros_hbm.at[pl.ds(0, CH)], slot)
        pltpu.sync_copy(row_hbm.at[wid], ridx_v)
        plsc.subcore_barrier()

        iota = lax.iota(jnp.int32, VEC)

        @pl.loop(0, rows_w // GRP)
        def _(gg):
            pltpu.sync_copy(
                cu_hbm.at[pl.ds((base + gg * GRP * CH) * 4, GRP * CH * 4)],
                cubuf)
            for j in range(GRP):
                g = gg * GRP + j
                pltpu.sync_copy(m_hbm.at[pl.ds(base + g * CH, CH)], mbuf)
                pltpu.sync_copy(mbuf, facc.at[ridx_v.at[g]], add=True)
                gv = jnp.full((VEC,), g, jnp.int32)
                for kk in range(CH * 4 // VEC):
                    flat = iota + kk * VEC
                    e_local = flat >> 2
                    comp = flat & 3
                    row = plsc.load_gather(ridx_v, [gv, e_local])
                    lane = ((row & 7) << 4) + comp
                    val = cubuf[pl.ds(j * CH * 4 + kk * VEC, VEC)]
                    plsc.store_scatter(slot, [e_local, lane], val)
                for kk in range(CH // VEC):
                    sl = pl.ds(kk * VEC, VEC)
                    rv = ridx_v[g, sl]
                    idx2[sl] = rv >> 3
                pltpu.sync_copy(slot, cacc.at[idx2], add=True)
                for kk in range(CH * 4 // VEC):
                    flat = iota + kk * VEC
                    e_local = flat >> 2
                    comp = flat & 3
                    row = plsc.load_gather(ridx_v, [gv, e_local])
                    lane = ((row & 7) << 4) + comp
                    plsc.store_scatter(slot, [e_local, lane],
                                       jnp.zeros((VEC,), jnp.float32))

        plsc.subcore_barrier()
        pltpu.sync_copy(facc.at[pl.ds(s * n_sub, n_sub)],
                        fout_hbm.at[c, pl.ds(s * n_sub, n_sub)])
        pltpu.sync_copy(cacc.at[pl.ds(s * nr_sub, nr_sub)],
                        cout_hbm.at[c, pl.ds(s * nr_sub, nr_sub)])

    return k(m, cu_flat, row3d, zeros)


def _rbf_expand(d0):
    ji = lax.broadcasted_iota(jnp.int32, (1, RBF_PAD), 1)
    j = ji.astype(jnp.float32)
    centers = j * (CUTOFF / (N_RBF - 1))
    lane_mask = (ji < N_RBF).astype(jnp.float32)
    width = CUTOFF / N_RBF
    rbf = jnp.exp(-((d0 - centers) ** 2) / width)
    cutoff_vals = 0.5 * (jnp.cos(math.pi * d0 / CUTOFF) + 1.0)
    cutoff_vals = cutoff_vals * (d0 < CUTOFF).astype(jnp.float32)
    return rbf * cutoff_vals * lane_mask


def _mlp_math(frow, fcol, ediff, ea, w1a, w1b, w1d, w1r, b1, w2, b2,
              cw1, cb1, cw2, cb2):
    ed = jnp.sum(ediff * ediff, axis=1, keepdims=True)

    dot = lambda a, b: jnp.dot(a, b, preferred_element_type=jnp.float32)
    pre = (dot(frow, w1a) + dot(fcol, w1b) + ed * w1d + dot(ea, w1r) + b1)
    m1 = pre * jax.nn.sigmoid(pre)
    h = dot(m1, w2) + b2
    m2 = h * jax.nn.sigmoid(h)
    c1p = dot(m2, cw1) + cb1
    c1 = c1p * jax.nn.sigmoid(c1p)
    cw = dot(c1, cw2) + cb2
    return m2, cw * ediff


def _mlp_body(frow_ref, fcol_ref, ediff_ref, ea_ref, w1a_ref, w1b_ref,
              w1d_ref, w1r_ref, b1_ref, w2_ref, b2_ref, cw1_ref, cb1_ref,
              cw2_ref, cb2_ref, m_ref, cu_ref):
    m, cu = _mlp_math(
        frow_ref[...], fcol_ref[...], ediff_ref[...], ea_ref[...],
        w1a_ref[...], w1b_ref[...], w1d_ref[...], w1r_ref[...], b1_ref[...],
        w2_ref[...], b2_ref[...], cw1_ref[...], cb1_ref[...], cw2_ref[...],
        cb2_ref[...])
    m_ref[...] = m
    cu_ref[...] = cu


def _mlp0_body(frow_ref, fcol_ref, ediff_ref, w1a_ref, w1b_ref, w1d_ref,
               w1r_ref, b1_ref, w2_ref, b2_ref, cw1_ref, cb1_ref, cw2_ref,
               cb2_ref, m_ref, cu_ref, ea_ref):
    ediff = ediff_ref[...]
    ed = jnp.sum(ediff * ediff, axis=1, keepdims=True)
    ea = _rbf_expand(jnp.sqrt(ed))
    ea_ref[...] = ea
    m, cu = _mlp_math(
        frow_ref[...], fcol_ref[...], ediff, ea,
        w1a_ref[...], w1b_ref[...], w1d_ref[...], w1r_ref[...], b1_ref[...],
        w2_ref[...], b2_ref[...], cw1_ref[...], cb1_ref[...], cw2_ref[...],
        cb2_ref[...])
    m_ref[...] = m
    cu_ref[...] = cu


def _edge_mlp(frow, fcol, ediff, ea, weights, block):
    E = frow.shape[0]
    em = lambda i: (i, 0)
    fixed = lambda i: (0, 0)
    wspecs = [
        pl.BlockSpec((H, H), fixed), pl.BlockSpec((H, H), fixed),
        pl.BlockSpec((1, H), fixed), pl.BlockSpec((RBF_PAD, H), fixed),
        pl.BlockSpec((1, H), fixed),
        pl.BlockSpec((H, H), fixed), pl.BlockSpec((1, H), fixed),
        pl.BlockSpec((H, H), fixed), pl.BlockSpec((1, H), fixed),
        pl.BlockSpec((H, 1), fixed), pl.BlockSpec((1, 1), fixed),
    ]
    out_m = [pl.BlockSpec((block, H), em), pl.BlockSpec((block, 4), em)]
    sds_m = [jax.ShapeDtypeStruct((E, H), jnp.float32),
             jax.ShapeDtypeStruct((E, 4), jnp.float32)]
    if ea is None:
        return pl.pallas_call(
            _mlp0_body,
            grid=(E // block,),
            in_specs=[pl.BlockSpec((block, H), em),
                      pl.BlockSpec((block, H), em),
                      pl.BlockSpec((block, 4), em)] + wspecs,
            out_specs=out_m + [pl.BlockSpec((block, RBF_PAD), em)],
            out_shape=sds_m + [jax.ShapeDtypeStruct((E, RBF_PAD),
                                                    jnp.float32)],
        )(frow, fcol, ediff, *weights)
    return pl.pallas_call(
        _mlp_body,
        grid=(E // block,),
        in_specs=[pl.BlockSpec((block, H), em), pl.BlockSpec((block, H), em),
                  pl.BlockSpec((block, 4), em),
                  pl.BlockSpec((block, RBF_PAD), em)] + wspecs,
        out_specs=out_m,
        out_shape=sds_m,
    )(frow, fcol, ediff, ea, *weights)


def _init_body(an_ref, emb_ref, F_ref):
    an = an_ref[...]
    types = lax.broadcasted_iota(jnp.int32, (1, H), 1)
    onehot = (an == types).astype(jnp.float32)
    F_ref[...] = jnp.dot(onehot, emb_ref[...],
                         preferred_element_type=jnp.float32)


def _init_F(an2d, emb_pad, block):
    N = an2d.shape[0]
    em = lambda i: (i, 0)
    fixed = lambda i: (0, 0)
    return pl.pallas_call(
        _init_body,
        grid=(N // block,),
        in_specs=[pl.BlockSpec((block, 1), em), pl.BlockSpec((H, H), fixed)],
        out_specs=pl.BlockSpec((block, H), em),
        out_shape=jax.ShapeDtypeStruct((N, H), jnp.float32),
    )(an2d, emb_pad)


def _upd_body(F_ref, fp0_ref, fp1_ref, C_ref, cp0_ref, cp1_ref,
              wt_ref, wb_ref, b_ref, Fo_ref, Co_ref):
    dot = lambda a, b: jnp.dot(a, b, preferred_element_type=jnp.float32)
    agg = fp0_ref[...] + fp1_ref[...]
    pre = (dot(F_ref[...], wt_ref[...]) + dot(agg, wb_ref[...]) + b_ref[...])
    Fo_ref[...] = pre * jax.nn.sigmoid(pre)
    cagg = cp0_ref[...] + cp1_ref[...]
    Co_ref[...] = C_ref[...] + cagg[:, :4]


def _node_update(F, fp0, fp1, C, cp0, cp1, wt, wb, b, block):
    N = F.shape[0]
    NP = C.shape[0]
    em = lambda i: (i, 0)
    fixed = lambda i: (0, 0)
    return pl.pallas_call(
        _upd_body,
        grid=(N // block,),
        in_specs=[pl.BlockSpec((block, H), em),
                  pl.BlockSpec((block, H), em), pl.BlockSpec((block, H), em),
                  pl.BlockSpec((block, 4), em),
                  pl.BlockSpec((block, 16), em), pl.BlockSpec((block, 16), em),
                  pl.BlockSpec((H, H), fixed), pl.BlockSpec((H, H), fixed),
                  pl.BlockSpec((1, H), fixed)],
        out_specs=[pl.BlockSpec((block, H), em), pl.BlockSpec((block, 4), em)],
        out_shape=[jax.ShapeDtypeStruct((N, H), jnp.float32),
                   jax.ShapeDtypeStruct((NP, 4), jnp.float32)],
    )(F, fp0, fp1, C, cp0, cp1, wt, wb, b)


def _head_body(F_ref, w1_ref, b1_ref, w2_ref, b2_ref, o_ref):
    dot = lambda a, b: jnp.dot(a, b, preferred_element_type=jnp.float32)
    h = dot(F_ref[...], w1_ref[...]) + b1_ref[...]
    h = h * jax.nn.sigmoid(h)
    e = dot(h, w2_ref[...]) + b2_ref[...]
    s = jnp.sum(e).reshape(1, 1)

    @pl.when(pl.program_id(0) == 0)
    def _():
        o_ref[...] = jnp.zeros_like(o_ref)

    o_ref[...] += s


def _head(F, w1, b1, w2, b2, block):
    N = F.shape[0]
    em = lambda i: (i, 0)
    fixed = lambda i: (0, 0)
    out = pl.pallas_call(
        _head_body,
        grid=(N // block,),
        in_specs=[pl.BlockSpec((block, H), em),
                  pl.BlockSpec((H, H), fixed), pl.BlockSpec((1, H), fixed),
                  pl.BlockSpec((H, 1), fixed), pl.BlockSpec((1, 1), fixed)],
        out_specs=pl.BlockSpec((1, 1), fixed),
        out_shape=jax.ShapeDtypeStruct((1, 1), jnp.float32),
    )(F, w1, b1, w2, b2)
    return out[0, 0]


def kernel(atomic_numbers, coords, edge_index, emb,
           msg_W1, msg_b1, msg_W2, msg_b2,
           coord_W1, coord_b1, coord_W2, coord_b2,
           feat_W, feat_b, ep_W1, ep_b1, ep_W2, ep_b2):
    N = coords.shape[0]
    E = edge_index.shape[1]
    n_layers = msg_W1.shape[0]
    e_block = 2560 if E % 2560 == 0 else 512
    n_block = 2000

    rows_w = E // NW // CH
    row3d = edge_index[0].reshape(NW, rows_w, CH)
    col3d = edge_index[1].reshape(NW, rows_w, CH)
    an2d = atomic_numbers.reshape(N, 1).astype(jnp.int32)
    emb_pad = jnp.pad(emb, ((0, H - emb.shape[0]), (0, 0)))
    NP = ((N + 1023) // 1024) * 1024
    zeros = jnp.zeros((NP, H), jnp.float32)

    F = _init_F(an2d, emb_pad, n_block)
    C = jnp.pad(coords, ((0, NP - N), (0, 1)))

    ea = None
    for i in range(n_layers):
        W1 = msg_W1[i]
        weights = (
            W1[:H], W1[H:2 * H], W1[2 * H:2 * H + 1],
            jnp.pad(W1[2 * H + 1:], ((0, RBF_PAD - N_RBF), (0, 0))),
            msg_b1[i].reshape(1, H),
            msg_W2[i], msg_b2[i].reshape(1, H),
            coord_W1[i], coord_b1[i].reshape(1, H),
            coord_W2[i], coord_b2[i].reshape(1, 1),
        )
        frow, fcol, ediff_flat = _sc_gather_geom(F, C.reshape(NP * 4),
                                                 row3d, col3d)
        ediff = ediff_flat.reshape(E, 4)
        if ea is None:
            m, cu, ea = _edge_mlp(frow, fcol, ediff, None, weights, e_block)
        else:
            m, cu = _edge_mlp(frow, fcol, ediff, ea, weights, e_block)
        fparts, cparts = _sc_scatter(m, cu.reshape(E * 4), row3d, zeros)
        cp = cparts.reshape(NCORES, NP, 16)
        F, C = _node_update(F, fparts[0], fparts[1], C, cp[0], cp[1],
                            feat_W[i][:H], feat_W[i][H:],
                            feat_b[i].reshape(1, H), n_block)

    return _head(F, ep_W1, ep_b1.reshape(1, H),
                 ep_W2, ep_b2.reshape(1, 1), n_block)
